# Initial kernel scaffold; baseline (speedup 1.0000x reference)
import jax
import jax.numpy as jnp
from jax.experimental import pallas as pl


def kernel(log_x_start, log_x_t, t_edge):
    raise NotImplementedError("write your pallas kernel here")


# speedup vs baseline: 21.9938x; 21.9938x over previous
import numpy as np
import jax
import jax.numpy as jnp
from jax.experimental import pallas as pl
from jax.experimental.pallas import tpu as pltpu

_N = 1000.0
_LOG_P1 = float(np.log(np.float32(1e-12)))
_LOG_N = float(np.log(np.float64(1000.0)))
_LOG_NP1 = float(np.log(np.float64(1001.0)))


def _lae(a, b):
    m = jnp.maximum(a, b)
    return m + jnp.log1p(jnp.exp(-jnp.abs(a - b)))


def _body(x_ref, q_ref, t_ref, o_ref):
    x = x_ref[...]
    q = q_ref[...]
    tf = t_ref[...].astype(jnp.float32)
    B, C = x.shape

    lane = jax.lax.broadcasted_iota(jnp.int32, (B, C), 1)
    even = (lane % 2) == 0

    def partner(v):
        return jnp.where(even, pltpu.roll(v, C - 1, 1), pltpu.roll(v, 1, 1))

    xp = partner(x)
    qp = partner(q)

    zs = _lae(x, xp)
    ls_own = x - zs
    ls_part = xp - zs
    zq = _lae(q, qp)
    lq_own = q - zq
    lq_part = qp - zq

    nt = _N - tf
    log_nt1 = jnp.log(nt + 1.0)
    la = jnp.log(nt) - log_nt1
    b = -log_nt1
    lca = jnp.where(tf >= 1.0, log_nt1, _LOG_N) - _LOG_NP1
    l1mca = jnp.log(jnp.maximum(tf, 1.0)) - _LOG_NP1
    bp = b + _LOG_P1

    a1 = jnp.where(even, bp + lq_part, la + lq_own)
    a2 = jnp.where(even, lq_own, b + lq_part)
    b1 = jnp.where(even, l1mca, lca + ls_own)
    b2 = jnp.where(even, lca + ls_own, l1mca + _LOG_P1)

    u = _lae(a1, a2) + _lae(b1, b2)
    up = partner(u)
    lse = _lae(u, up)
    o_ref[...] = u - lse


def kernel(log_x_start, log_x_t, t_edge):
    E = log_x_start.shape[0]
    C = 256
    R = (2 * E) // C
    X = log_x_start.reshape(R, C)
    Q = log_x_t.reshape(R, C)
    T = jnp.repeat(t_edge.astype(jnp.int32), 2).reshape(R, C)

    grid = 5 if R % 5 == 0 else 1
    br = R // grid
    spec = pl.BlockSpec((br, C), lambda i: (i, 0))
    o = pl.pallas_call(
        _body,
        grid=(grid,),
        in_specs=[spec] * 3,
        out_specs=spec,
        out_shape=jax.ShapeDtypeStruct((R, C), jnp.float32),
    )(X, Q, T)
    return o.reshape(E, 2)
